# Initial kernel scaffold; baseline (speedup 1.0000x reference)
import jax
import jax.numpy as jnp
from jax.experimental import pallas as pl


def kernel(x):
    raise NotImplementedError("write your pallas kernel here")


# speedup vs baseline: 1.3669x; 1.3669x over previous
import functools

import numpy as np
import jax
import jax.numpy as jnp
from jax import lax
from jax.experimental import pallas as pl
from jax.experimental.pallas import tpu as pltpu
from jax.experimental.pallas import tpu_sc as plsc

R = 64
N = 8192
L = 16
CHUNK = N // L
NW = 32
ROWS_PER_W = R // NW
BINS = 256
INT_MIN = np.int32(-(2 ** 31))


def _fwd_key(b):
    m = lax.shift_right_arithmetic(b, 31)
    return b ^ (m | INT_MIN)


def _inv_key(k):
    m = lax.shift_right_arithmetic(k, 31)
    return k ^ ((~m) | INT_MIN)


def _sort_body(x_hbm, out_hbm, dataf, keys0, keys1, run):
    wid = lax.axis_index("s") * 2 + lax.axis_index("c")
    lane = lax.iota(jnp.int32, L)
    ones = jnp.ones((L,), jnp.int32)
    zeros = jnp.zeros((L,), jnp.int32)

    def sort_one_row(row):
        pltpu.sync_copy(x_hbm.at[row], dataf)

        for p in range(4):
            shift = 8 * p
            src = (dataf, keys0, keys1, keys0)[p]
            dst = (keys0, keys1, keys0, dataf)[p]

            def load_key(j):
                idx_t = lane * CHUNK + j
                if p == 0:
                    v = plsc.load_gather(dataf, [idx_t])
                    return _fwd_key(plsc.bitcast(v, jnp.int32))
                return plsc.load_gather(src, [idx_t])

            def digit(k):
                if shift:
                    k = lax.shift_right_logical(k, shift)
                return k & 0xFF

            def zero_body(i, c):
                run[pl.ds(i * L, L)] = zeros
                return c

            lax.fori_loop(0, BINS, zero_body, 0)

            def hist_body(j, c):
                d = digit(load_key(j))
                plsc.addupdate_scatter(run, [d * L + lane], ones)
                return c

            lax.fori_loop(0, CHUNK, hist_body, 0)

            def scan_body(i, carry):
                v = run[pl.ds(i * L, L)]
                cs = plsc.cumsum(v)
                run[pl.ds(i * L, L)] = cs - v + carry
                return carry + jnp.sum(v)

            lax.fori_loop(0, BINS, scan_body, jnp.int32(0))

            def scat_body(j, c):
                k = load_key(j)
                hidx = digit(k) * L + lane
                pos = plsc.load_gather(run, [hidx])
                if p == 3:
                    vout = plsc.bitcast(_inv_key(k), jnp.float32)
                    plsc.store_scatter(dataf, [pos], vout)
                else:
                    plsc.store_scatter(dst, [pos], k)
                plsc.addupdate_scatter(run, [hidx], ones)
                return c

            lax.fori_loop(0, CHUNK, scat_body, 0)

        pltpu.sync_copy(dataf, out_hbm.at[row])

    for rr in range(ROWS_PER_W):
        sort_one_row(wid * ROWS_PER_W + rr)


_sc_sort = functools.partial(
    pl.kernel,
    out_type=jax.ShapeDtypeStruct((R, N), jnp.float32),
    mesh=plsc.VectorSubcoreMesh(core_axis_name="c", subcore_axis_name="s"),
    compiler_params=pltpu.CompilerParams(needs_layout_passes=False),
    scratch_types=[
        pltpu.VMEM((N,), jnp.float32),
        pltpu.VMEM((N,), jnp.int32),
        pltpu.VMEM((N,), jnp.int32),
        pltpu.VMEM((BINS * L,), jnp.int32),
    ],
)(_sort_body)


@jax.jit
def kernel(x):
    return _sc_sort(x)


# speedup vs baseline: 1.3679x; 1.0007x over previous
import functools

import numpy as np
import jax
import jax.numpy as jnp
from jax import lax
from jax.experimental import pallas as pl
from jax.experimental.pallas import tpu as pltpu
from jax.experimental.pallas import tpu_sc as plsc

R = 64
N = 8192
L = 16
CHUNK = N // L
NW = 32
ROWS_PER_W = R // NW
BINS = 256
INT_MIN = np.int32(-(2 ** 31))


def _fwd_key(b):
    m = lax.shift_right_arithmetic(b, 31)
    return b ^ (m | INT_MIN)


def _inv_key(k):
    m = lax.shift_right_arithmetic(k, 31)
    return k ^ ((~m) | INT_MIN)


def _sort_body(x_hbm, out_hbm, dataf0, dataf1, k0a, k1a, k0b, k1b, run0, run1):
    wid = lax.axis_index("s") * 2 + lax.axis_index("c")
    lane = lax.iota(jnp.int32, L)
    ones = jnp.ones((L,), jnp.int32)
    zeros = jnp.zeros((L,), jnp.int32)

    dataf = (dataf0, dataf1)
    keys = ((k0a, k1a), (k0b, k1b))
    run = (run0, run1)
    rows = (wid * ROWS_PER_W, wid * ROWS_PER_W + 1)

    for r in range(2):
        pltpu.sync_copy(x_hbm.at[rows[r]], dataf[r])

    for p in range(4):
        shift = 8 * p

        def src(r):
            return (dataf[r], keys[r][0], keys[r][1], keys[r][0])[p]

        def dst(r):
            return (keys[r][0], keys[r][1], keys[r][0], dataf[r])[p]

        def load_key(r, j):
            idx_t = lane * CHUNK + j
            if p == 0:
                v = plsc.load_gather(dataf[r], [idx_t])
                return _fwd_key(plsc.bitcast(v, jnp.int32))
            return plsc.load_gather(src(r), [idx_t])

        def digit(k):
            if shift:
                k = lax.shift_right_logical(k, shift)
            return k & 0xFF

        def zero_body(i, c):
            for r in range(2):
                run[r][pl.ds(i * L, L)] = zeros
            return c

        lax.fori_loop(0, BINS, zero_body, 0)

        def hist_body(j, c):
            for r in range(2):
                d = digit(load_key(r, j))
                plsc.addupdate_scatter(run[r], [d * L + lane], ones)
            return c

        lax.fori_loop(0, CHUNK, hist_body, 0)

        def scan_body(i, carry):
            nxt = []
            for r in range(2):
                v = run[r][pl.ds(i * L, L)]
                cs = plsc.cumsum(v)
                run[r][pl.ds(i * L, L)] = cs - v + carry[r]
                nxt.append(carry[r] + jnp.sum(v))
            return tuple(nxt)

        lax.fori_loop(0, BINS, scan_body, (jnp.int32(0), jnp.int32(0)))

        def scat_body(j, c):
            for r in range(2):
                k = load_key(r, j)
                hidx = digit(k) * L + lane
                pos = plsc.load_gather(run[r], [hidx])
                if p == 3:
                    vout = plsc.bitcast(_inv_key(k), jnp.float32)
                    plsc.store_scatter(dataf[r], [pos], vout)
                else:
                    plsc.store_scatter(dst(r), [pos], k)
                plsc.addupdate_scatter(run[r], [hidx], ones)
            return c

        lax.fori_loop(0, CHUNK, scat_body, 0)

    for r in range(2):
        pltpu.sync_copy(dataf[r], out_hbm.at[rows[r]])


_sc_sort = functools.partial(
    pl.kernel,
    out_type=jax.ShapeDtypeStruct((R, N), jnp.float32),
    mesh=plsc.VectorSubcoreMesh(core_axis_name="c", subcore_axis_name="s"),
    compiler_params=pltpu.CompilerParams(needs_layout_passes=False),
    scratch_types=[
        pltpu.VMEM((N,), jnp.float32),
        pltpu.VMEM((N,), jnp.float32),
        pltpu.VMEM((N,), jnp.int32),
        pltpu.VMEM((N,), jnp.int32),
        pltpu.VMEM((N,), jnp.int32),
        pltpu.VMEM((N,), jnp.int32),
        pltpu.VMEM((BINS * L,), jnp.int32),
        pltpu.VMEM((BINS * L,), jnp.int32),
    ],
)(_sort_body)


@jax.jit
def kernel(x):
    return _sc_sort(x)


# speedup vs baseline: 1.6373x; 1.1970x over previous
import functools

import numpy as np
import jax
import jax.numpy as jnp
from jax import lax
from jax.experimental import pallas as pl
from jax.experimental.pallas import tpu as pltpu
from jax.experimental.pallas import tpu_sc as plsc

R = 64
N = 8192
L = 16
CHUNK = N // L
U = 4
JB = CHUNK // U
NW = 32
ROWS_PER_W = R // NW
BINS = 256
INT_MIN = np.int32(-(2 ** 31))


def _fwd_key(b):
    m = lax.shift_right_arithmetic(b, 31)
    return b ^ (m | INT_MIN)


def _inv_key(k):
    m = lax.shift_right_arithmetic(k, 31)
    return k ^ ((~m) | INT_MIN)


def _sort_body(x_hbm, out_hbm, *scratch):
    dataf = scratch[0:2]
    keys = ((scratch[2], scratch[3]), (scratch[4], scratch[5]))
    hist = scratch[6:8]
    offs = (scratch[8:12], scratch[12:16])

    wid = lax.axis_index("s") * 2 + lax.axis_index("c")
    lane = lax.iota(jnp.int32, L)
    ones = jnp.ones((L,), jnp.int32)
    zeros = jnp.zeros((L,), jnp.int32)
    lane9 = lane * CHUNK
    rows = (wid * ROWS_PER_W, wid * ROWS_PER_W + 1)

    for r in range(2):
        pltpu.sync_copy(x_hbm.at[rows[r]], dataf[r])

    def zero_body(i, c):
        for r in range(2):
            for u in range(U):
                hist[r][pl.ds((u * BINS + i) * L, L)] = zeros
        return c

    lax.fori_loop(0, BINS, zero_body, 0)

    for p in range(4):
        shift = 8 * p

        def src(r):
            return (dataf[r], keys[r][0], keys[r][1], keys[r][0])[p]

        def dst(r):
            return (keys[r][0], keys[r][1], keys[r][0], dataf[r])[p]

        def to_key(v):
            if p == 0:
                return _fwd_key(plsc.bitcast(v, jnp.int32))
            return v

        def digit(k):
            if shift:
                k = lax.shift_right_logical(k, shift)
            return k & 0xFF

        def hist_body(i, c):
            l_s = lax.shift_right_logical(i, 3)
            w_s = i & 7
            for r in range(2):
                for u in range(U):
                    base = l_s * CHUNK + u * JB + w_s * L
                    v = src(r)[pl.ds(base, L)]
                    d = digit(to_key(v))
                    hidx = u * (BINS * L) + lax.shift_left(d, 4) + l_s
                    plsc.addupdate_scatter(hist[r], [hidx], ones)
            return c

        lax.fori_loop(0, L * (CHUNK // U // L), hist_body, 0)

        def scan_body(i, carry):
            nxt = []
            for r in range(2):
                vs = []
                for u in range(U):
                    vs.append(hist[r][pl.ds((u * BINS + i) * L, L)])
                for u in range(U):
                    hist[r][pl.ds((u * BINS + i) * L, L)] = zeros
                t = vs[0] + vs[1] + vs[2] + vs[3]
                cs = plsc.cumsum(t)
                excl = cs - t + carry[r]
                acc = excl
                for u in range(U):
                    offs[r][u][pl.ds(i * L, L)] = acc
                    if u < U - 1:
                        acc = acc + vs[u]
                nxt.append(carry[r] + jnp.sum(t))
            return tuple(nxt)

        lax.fori_loop(0, BINS, scan_body, (jnp.int32(0), jnp.int32(0)))

        def scat_body(j, c):
            for r in range(2):
                for u in range(U):
                    idx_t = lane9 + (u * JB + j)
                    k = to_key(plsc.load_gather(src(r), [idx_t]))
                    hidx = lax.shift_left(digit(k), 4) + lane
                    pos = plsc.load_gather(offs[r][u], [hidx])
                    if p == 3:
                        vout = plsc.bitcast(_inv_key(k), jnp.float32)
                        plsc.store_scatter(dataf[r], [pos], vout)
                    else:
                        plsc.store_scatter(dst(r), [pos], k)
                    plsc.addupdate_scatter(offs[r][u], [hidx], ones)
            return c

        lax.fori_loop(0, JB, scat_body, 0)

    for r in range(2):
        pltpu.sync_copy(dataf[r], out_hbm.at[rows[r]])


_sc_sort = functools.partial(
    pl.kernel,
    out_type=jax.ShapeDtypeStruct((R, N), jnp.float32),
    mesh=plsc.VectorSubcoreMesh(core_axis_name="c", subcore_axis_name="s"),
    compiler_params=pltpu.CompilerParams(needs_layout_passes=False),
    scratch_types=[
        pltpu.VMEM((N,), jnp.float32),
        pltpu.VMEM((N,), jnp.float32),
        pltpu.VMEM((N,), jnp.int32),
        pltpu.VMEM((N,), jnp.int32),
        pltpu.VMEM((N,), jnp.int32),
        pltpu.VMEM((N,), jnp.int32),
        pltpu.VMEM((U * BINS * L,), jnp.int32),
        pltpu.VMEM((U * BINS * L,), jnp.int32),
    ] + [pltpu.VMEM((BINS * L,), jnp.int32) for _ in range(2 * U)],
)(_sort_body)


@jax.jit
def kernel(x):
    return _sc_sort(x)


# speedup vs baseline: 2.7094x; 1.6548x over previous
import functools

import numpy as np
import jax
import jax.numpy as jnp
from jax import lax
from jax.experimental import pallas as pl
from jax.experimental.pallas import tpu as pltpu
from jax.experimental.pallas import tpu_sc as plsc

R = 64
N = 8192
L = 16
CHUNK = N // L
U = 4
JB = CHUNK // U
NW = 32
ROWS_PER_W = R // NW
BINS = 256
INT_MIN = np.int32(-(2 ** 31))


def _sort_body(x_hbm, out_hbm, *scratch):
    dataf = scratch[0:2]
    keys = ((scratch[2], scratch[3]), (scratch[4], scratch[5]))
    hist = scratch[6:8]
    offs = (scratch[8:12], scratch[12:16])

    wid = lax.axis_index("s") * 2 + lax.axis_index("c")
    lane = lax.iota(jnp.int32, L)
    ones = jnp.ones((L,), jnp.int32)
    zeros = jnp.zeros((L,), jnp.int32)
    zvec = jnp.zeros((L,), jnp.int32)
    fifteen = jnp.full((L,), 15, jnp.int32)
    lane9 = lane * CHUNK
    rows = (wid * ROWS_PER_W, wid * ROWS_PER_W + 1)
    SU = [(r, u) for r in range(2) for u in range(U)]

    for r in range(2):
        pltpu.sync_copy(x_hbm.at[rows[r]], dataf[r])

    def zero_body(i, c):
        for r in range(2):
            for u in range(U):
                hist[r][pl.ds((u * BINS + i) * L, L)] = zeros
        return c

    lax.fori_loop(0, BINS, zero_body, 0)

    for p in range(4):
        shift = 8 * p

        def a_src(r):
            return (dataf[r], keys[r][0], keys[r][1], keys[r][0])[p]

        def c_src(r):
            return (keys[r][1], keys[r][0], keys[r][1], keys[r][0])[p]

        def c_dst(r):
            return (keys[r][0], keys[r][1], keys[r][0], dataf[r])[p]

        def hmask(ks):
            if shift >= 4:
                t = [lax.shift_right_logical(k, shift - 4) for k in ks]
            else:
                t = [lax.shift_left(k, 4) for k in ks]
            return [t_ & 0xFF0 for t_ in t]

        def hist_body(i, c):
            l_s = lax.shift_right_logical(i, 3)
            w_s = i & 7
            bases = [l_s * CHUNK + u * JB + w_s * L for u in range(U)]
            vs = [a_src(r)[pl.ds(bases[u], L)] for (r, u) in SU]
            if p == 0:
                bs = [plsc.bitcast(v, jnp.int32) for v in vs]
                ms = [lax.shift_right_arithmetic(b, 31) for b in bs]
                ms = [m | INT_MIN for m in ms]
                ks = [b ^ m for b, m in zip(bs, ms)]
                for (r, u), k in zip(SU, ks):
                    keys[r][1][pl.ds(bases[u], L)] = k
            else:
                ks = vs
            hs = hmask(ks)
            hidxs = [h | l_s for h in hs]
            for (r, u), h in zip(SU, hidxs):
                plsc.addupdate_scatter(
                    hist[r], [h + np.int32(u * BINS * L)], ones)
            return c

        lax.fori_loop(0, CHUNK // U // L * L, hist_body, 0)

        def scan_body(i, carry):
            vs = [[hist[r][pl.ds((u * BINS + i) * L, L)] for u in range(U)]
                  for r in range(2)]
            for r in range(2):
                for u in range(U):
                    hist[r][pl.ds((u * BINS + i) * L, L)] = zeros
            t01 = [(v[0] + v[1], v[2] + v[3]) for v in vs]
            ts = [a + b for a, b in t01]
            css = [plsc.cumsum(t) for t in ts]
            excls = [cs - t + cry for cs, t, cry in zip(css, ts, carry)]
            tops = [jnp.take(cs, fifteen) for cs in css]
            nxt = tuple(cry + top for cry, top in zip(carry, tops))
            for r in range(2):
                acc = excls[r]
                for u in range(U):
                    offs[r][u][pl.ds(i * L, L)] = acc
                    if u < U - 1:
                        acc = acc + vs[r][u]
            return nxt

        lax.fori_loop(0, BINS, scan_body, (zvec, zvec))

        def scat_body(j, c):
            idxs = [lane9 + (u * JB + j) for u in range(U)]
            ks = [plsc.load_gather(c_src(r), [idxs[u]]) for (r, u) in SU]
            hs = hmask(ks)
            hidxs = [h | lane for h in hs]
            poss = [plsc.load_gather(offs[r][u], [h])
                    for (r, u), h in zip(SU, hidxs)]
            if p == 3:
                ms = [lax.shift_right_arithmetic(k, 31) for k in ks]
                ms = [(~m) | INT_MIN for m in ms]
                outs = [plsc.bitcast(k ^ m, jnp.float32)
                        for k, m in zip(ks, ms)]
            else:
                outs = ks
            for (r, u), pos, o in zip(SU, poss, outs):
                plsc.store_scatter(c_dst(r), [pos], o)
            for (r, u), h in zip(SU, hidxs):
                plsc.addupdate_scatter(offs[r][u], [h], ones)
            return c

        lax.fori_loop(0, JB, scat_body, 0)

    for r in range(2):
        pltpu.sync_copy(dataf[r], out_hbm.at[rows[r]])


_sc_sort = functools.partial(
    pl.kernel,
    out_type=jax.ShapeDtypeStruct((R, N), jnp.float32),
    mesh=plsc.VectorSubcoreMesh(core_axis_name="c", subcore_axis_name="s"),
    compiler_params=pltpu.CompilerParams(needs_layout_passes=False),
    scratch_types=[
        pltpu.VMEM((N,), jnp.float32),
        pltpu.VMEM((N,), jnp.float32),
        pltpu.VMEM((N,), jnp.int32),
        pltpu.VMEM((N,), jnp.int32),
        pltpu.VMEM((N,), jnp.int32),
        pltpu.VMEM((N,), jnp.int32),
        pltpu.VMEM((U * BINS * L,), jnp.int32),
        pltpu.VMEM((U * BINS * L,), jnp.int32),
    ] + [pltpu.VMEM((BINS * L,), jnp.int32) for _ in range(2 * U)],
)(_sort_body)


@jax.jit
def kernel(x):
    return _sc_sort(x)


# speedup vs baseline: 3.9656x; 1.4636x over previous
import functools

import numpy as np
import jax
import jax.numpy as jnp
from jax import lax
from jax.experimental import pallas as pl
from jax.experimental.pallas import tpu as pltpu
from jax.experimental.pallas import tpu_sc as plsc

R = 64
N = 8192
L = 16
CHUNK = N // L
U = 4
JB = CHUNK // U
NW = 32
ROWS_PER_W = R // NW
BINS = 256
INT_MIN = np.int32(-(2 ** 31))


def _sort_body(x_hbm, out_hbm, *scratch):
    dataf = scratch[0:2]
    keys = ((scratch[2], scratch[3]), (scratch[4], scratch[5]))
    hist = scratch[6:8]
    offs = (scratch[8:12], scratch[12:16])

    wid = lax.axis_index("s") * 2 + lax.axis_index("c")
    lane = lax.iota(jnp.int32, L)
    ones = jnp.ones((L,), jnp.int32)
    zeros = jnp.zeros((L,), jnp.int32)
    zvec = jnp.zeros((L,), jnp.int32)
    fifteen = jnp.full((L,), 15, jnp.int32)
    lane9 = lane * CHUNK + lane
    rows = (wid * ROWS_PER_W, wid * ROWS_PER_W + 1)
    SU = [(r, u) for r in range(2) for u in range(U)]

    for r in range(2):
        pltpu.sync_copy(x_hbm.at[rows[r]], dataf[r])

    def zero_body(i, c):
        for r in range(2):
            for u in range(U):
                hist[r][pl.ds((u * BINS + i) * L, L)] = zeros
        return c

    lax.fori_loop(0, BINS, zero_body, 0)

    for p in range(4):
        shift = 8 * p

        def a_src(r):
            return (dataf[r], keys[r][0], keys[r][1], keys[r][0])[p]

        def c_src(r):
            return (keys[r][1], keys[r][0], keys[r][1], keys[r][0])[p]

        def c_dst(r):
            return (keys[r][0], keys[r][1], keys[r][0], dataf[r])[p]

        def hmask(ks):
            if shift >= 4:
                t = [lax.shift_right_logical(k, shift - 4) for k in ks]
            else:
                t = [lax.shift_left(k, 4) for k in ks]
            return [t_ & 0xFF0 for t_ in t]

        def hist_body(i, c):
            l_s = lax.shift_right_logical(i, 3)
            w_s = i & 7
            bases = [l_s * CHUNK + u * JB + w_s * L for u in range(U)]
            sk_bases = [b + l_s for b in bases]
            if p == 0:
                vs = [a_src(r)[pl.ds(bases[u], L)] for (r, u) in SU]
                bs = [plsc.bitcast(v, jnp.int32) for v in vs]
                ms = [lax.shift_right_arithmetic(b, 31) for b in bs]
                ms = [m | INT_MIN for m in ms]
                ks = [b ^ m for b, m in zip(bs, ms)]
                for (r, u), k in zip(SU, ks):
                    keys[r][1][pl.ds(sk_bases[u], L)] = k
            else:
                ks = [a_src(r)[pl.ds(sk_bases[u], L)] for (r, u) in SU]
            hs = hmask(ks)
            hidxs = [h | l_s for h in hs]
            for (r, u), h in zip(SU, hidxs):
                plsc.addupdate_scatter(
                    hist[r], [h + np.int32(u * BINS * L)], ones)
            return c

        lax.fori_loop(0, CHUNK // U // L * L, hist_body, 0)

        def scan_body(i, carry):
            vs = [[hist[r][pl.ds((u * BINS + i) * L, L)] for u in range(U)]
                  for r in range(2)]
            for r in range(2):
                for u in range(U):
                    hist[r][pl.ds((u * BINS + i) * L, L)] = zeros
            t01 = [(v[0] + v[1], v[2] + v[3]) for v in vs]
            ts = [a + b for a, b in t01]
            css = [plsc.cumsum(t) for t in ts]
            excls = [cs - t + cry for cs, t, cry in zip(css, ts, carry)]
            tops = [jnp.take(cs, fifteen) for cs in css]
            nxt = tuple(cry + top for cry, top in zip(carry, tops))
            for r in range(2):
                acc = excls[r]
                for u in range(U):
                    offs[r][u][pl.ds(i * L, L)] = acc
                    if u < U - 1:
                        acc = acc + vs[r][u]
            return nxt

        lax.fori_loop(0, BINS, scan_body, (zvec, zvec))

        def scat_body(j, c):
            idxs = [lane9 + (u * JB + j) for u in range(U)]
            ks = [plsc.load_gather(c_src(r), [idxs[u]]) for (r, u) in SU]
            hs = hmask(ks)
            hidxs = [h | lane for h in hs]
            poss = [plsc.load_gather(offs[r][u], [h])
                    for (r, u), h in zip(SU, hidxs)]
            if p == 3:
                ms = [lax.shift_right_arithmetic(k, 31) for k in ks]
                ms = [(~m) | INT_MIN for m in ms]
                outs = [plsc.bitcast(k ^ m, jnp.float32)
                        for k, m in zip(ks, ms)]
            else:
                outs = ks
                poss = [pos + lax.shift_right_logical(pos, 9) for pos in poss]
            for (r, u), pos, o in zip(SU, poss, outs):
                plsc.store_scatter(c_dst(r), [pos], o)
            for (r, u), h in zip(SU, hidxs):
                plsc.addupdate_scatter(offs[r][u], [h], ones)
            return c

        lax.fori_loop(0, JB, scat_body, 0)

    for r in range(2):
        pltpu.sync_copy(dataf[r], out_hbm.at[rows[r]])


_sc_sort = functools.partial(
    pl.kernel,
    out_type=jax.ShapeDtypeStruct((R, N), jnp.float32),
    mesh=plsc.VectorSubcoreMesh(core_axis_name="c", subcore_axis_name="s"),
    compiler_params=pltpu.CompilerParams(needs_layout_passes=False),
    scratch_types=[
        pltpu.VMEM((N,), jnp.float32),
        pltpu.VMEM((N,), jnp.float32),
        pltpu.VMEM((N + L,), jnp.int32),
        pltpu.VMEM((N + L,), jnp.int32),
        pltpu.VMEM((N + L,), jnp.int32),
        pltpu.VMEM((N + L,), jnp.int32),
        pltpu.VMEM((U * BINS * L,), jnp.int32),
        pltpu.VMEM((U * BINS * L,), jnp.int32),
    ] + [pltpu.VMEM((BINS * L,), jnp.int32) for _ in range(2 * U)],
)(_sort_body)


@jax.jit
def kernel(x):
    return _sc_sort(x)


# speedup vs baseline: 5.3599x; 1.3516x over previous
import functools

import numpy as np
import jax
import jax.numpy as jnp
from jax import lax
from jax.experimental import pallas as pl
from jax.experimental.pallas import tpu as pltpu
from jax.experimental.pallas import tpu_sc as plsc

R = 64
N = 8192
L = 16
CHUNK = N // L
U = 4
JB = CHUNK // U
NW = 32
ROWS_PER_W = R // NW
BINS = 256
INT_MIN = np.int32(-(2 ** 31))


def _sort_body(x_hbm, out_hbm, *scratch):
    dataf = scratch[0:2]
    keys = ((scratch[2], scratch[3]), (scratch[4], scratch[5]))
    hist = scratch[6:8]
    offs = (scratch[8:12], scratch[12:16])
    sems = scratch[16:18]

    wid = lax.axis_index("s") * 2 + lax.axis_index("c")
    lane = lax.iota(jnp.int32, L)
    ones = jnp.ones((L,), jnp.int32)
    zeros = jnp.zeros((L,), jnp.int32)
    zvec = jnp.zeros((L,), jnp.int32)
    fifteen = jnp.full((L,), 15, jnp.int32)
    lane9 = lane * CHUNK + lane
    rows = (wid * ROWS_PER_W, wid * ROWS_PER_W + 1)
    SU = [(r, u) for r in range(2) for u in range(U)]

    cin = [pltpu.async_copy(x_hbm.at[rows[r]], dataf[r], sems[r])
           for r in range(2)]

    def zero_body(i, c):
        for r in range(2):
            for u in range(U):
                hist[r][pl.ds((u * BINS + i) * L, L)] = zeros
        return c

    lax.fori_loop(0, BINS, zero_body, 0)
    for c in cin:
        c.wait()

    for p in range(4):
        shift = 8 * p

        def c_src(r):
            return (keys[r][1], keys[r][0], keys[r][1], keys[r][0])[p]

        def c_dst(r):
            return (keys[r][0], keys[r][1], keys[r][0], dataf[r])[p]

        def hmask(ks, sh):
            if sh >= 4:
                t = [lax.shift_right_logical(k, sh - 4) for k in ks]
            else:
                t = [lax.shift_left(k, 4) for k in ks]
            return [t_ & 0xFF0 for t_ in t]

        if p == 0:
            def hist_body(i, c):
                l_s = lax.shift_right_logical(i, 3)
                w_s = i & 7
                bases = [l_s * CHUNK + u * JB + w_s * L for u in range(U)]
                vs = [dataf[r][pl.ds(bases[u], L)] for (r, u) in SU]
                bs = [plsc.bitcast(v, jnp.int32) for v in vs]
                ms = [lax.shift_right_arithmetic(b, 31) for b in bs]
                ms = [m | INT_MIN for m in ms]
                ks = [b ^ m for b, m in zip(bs, ms)]
                for (r, u), k in zip(SU, ks):
                    keys[r][1][pl.ds(bases[u] + l_s, L)] = k
                hs = hmask(ks, 0)
                hidxs = [h | l_s for h in hs]
                for (r, u), h in zip(SU, hidxs):
                    plsc.addupdate_scatter(
                        hist[r], [h + np.int32(u * BINS * L)], ones)
                return c

            lax.fori_loop(0, CHUNK // U, hist_body, 0)

        def scan_body(i, carry):
            vs = [[hist[r][pl.ds((u * BINS + i) * L, L)] for u in range(U)]
                  for r in range(2)]
            for r in range(2):
                for u in range(U):
                    hist[r][pl.ds((u * BINS + i) * L, L)] = zeros
            t01 = [(v[0] + v[1], v[2] + v[3]) for v in vs]
            ts = [a + b for a, b in t01]
            css = [plsc.cumsum(t) for t in ts]
            excls = [cs - t + cry for cs, t, cry in zip(css, ts, carry)]
            tops = [jnp.take(cs, fifteen) for cs in css]
            nxt = tuple(cry + top for cry, top in zip(carry, tops))
            for r in range(2):
                acc = excls[r]
                for u in range(U):
                    offs[r][u][pl.ds(i * L, L)] = acc
                    if u < U - 1:
                        acc = acc + vs[r][u]
            return nxt

        lax.fori_loop(0, BINS, scan_body, (zvec, zvec))

        def scat_body(j, c):
            idxs = [lane9 + (u * JB + j) for u in range(U)]
            ks = [plsc.load_gather(c_src(r), [idxs[u]]) for (r, u) in SU]
            hs = hmask(ks, shift)
            hidxs = [h | lane for h in hs]
            poss = [plsc.load_gather(offs[r][u], [h])
                    for (r, u), h in zip(SU, hidxs)]
            if p == 3:
                ms = [lax.shift_right_arithmetic(k, 31) for k in ks]
                ms = [(~m) | INT_MIN for m in ms]
                outs = [plsc.bitcast(k ^ m, jnp.float32)
                        for k, m in zip(ks, ms)]
                st_poss = poss
            else:
                outs = ks
                st_poss = [pos + lax.shift_right_logical(pos, 9)
                           for pos in poss]
            for (r, u), pos, o in zip(SU, st_poss, outs):
                plsc.store_scatter(c_dst(r), [pos], o)
            for (r, u), h in zip(SU, hidxs):
                plsc.addupdate_scatter(offs[r][u], [h], ones)
            if p < 3:
                ups = [lax.shift_left(pos, 5) & 0x3000 for pos in poss]
                lps = [lax.shift_right_logical(pos, 9) for pos in poss]
                dps = hmask(ks, shift + 8)
                h1 = [a | b for a, b in zip(ups, lps)]
                h2 = [a | b for a, b in zip(h1, dps)]
                for (r, u), h in zip(SU, h2):
                    plsc.addupdate_scatter(hist[r], [h], ones)
            return c

        lax.fori_loop(0, JB, scat_body, 0)

    cout = [pltpu.async_copy(dataf[r], out_hbm.at[rows[r]], sems[r])
            for r in range(2)]
    for c in cout:
        c.wait()


_sc_sort = functools.partial(
    pl.kernel,
    out_type=jax.ShapeDtypeStruct((R, N), jnp.float32),
    mesh=plsc.VectorSubcoreMesh(core_axis_name="c", subcore_axis_name="s"),
    compiler_params=pltpu.CompilerParams(needs_layout_passes=False),
    scratch_types=[
        pltpu.VMEM((N,), jnp.float32),
        pltpu.VMEM((N,), jnp.float32),
        pltpu.VMEM((N + L,), jnp.int32),
        pltpu.VMEM((N + L,), jnp.int32),
        pltpu.VMEM((N + L,), jnp.int32),
        pltpu.VMEM((N + L,), jnp.int32),
        pltpu.VMEM((U * BINS * L,), jnp.int32),
        pltpu.VMEM((U * BINS * L,), jnp.int32),
    ] + [pltpu.VMEM((BINS * L,), jnp.int32) for _ in range(2 * U)]
      + [pltpu.SemaphoreType.DMA, pltpu.SemaphoreType.DMA],
)(_sort_body)


@jax.jit
def kernel(x):
    return _sc_sort(x)


# speedup vs baseline: 5.3603x; 1.0001x over previous
import functools

import numpy as np
import jax
import jax.numpy as jnp
from jax import lax
from jax.experimental import pallas as pl
from jax.experimental.pallas import tpu as pltpu
from jax.experimental.pallas import tpu_sc as plsc

R = 64
N = 8192
L = 16
CHUNK = N // L
U = 4
JB = CHUNK // U
NW = 32
ROWS_PER_W = R // NW
BINS = 256
INT_MIN = np.int32(-(2 ** 31))


def _sort_body(x_hbm, out_hbm, *scratch):
    dataf = scratch[0:2]
    keys = ((scratch[2], scratch[3]), (scratch[4], scratch[5]))
    hist = scratch[6:8]
    offs = (scratch[8:12], scratch[12:16])
    sems = scratch[16:18]

    wid = lax.axis_index("s") * 2 + lax.axis_index("c")
    lane = lax.iota(jnp.int32, L)
    ones = jnp.ones((L,), jnp.int32)
    zeros = jnp.zeros((L,), jnp.int32)
    zvec = jnp.zeros((L,), jnp.int32)
    fifteen = jnp.full((L,), 15, jnp.int32)
    lane9 = lane * CHUNK + lane
    rows = (wid * ROWS_PER_W, wid * ROWS_PER_W + 1)
    SU = [(r, u) for r in range(2) for u in range(U)]

    cin = [pltpu.async_copy(x_hbm.at[rows[r]], dataf[r], sems[r])
           for r in range(2)]

    def zero_body(i, c):
        for r in range(2):
            for u in range(U):
                hist[r][pl.ds((u * BINS + i) * L, L)] = zeros
        return c

    lax.fori_loop(0, BINS, zero_body, 0)
    for c in cin:
        c.wait()

    for p in range(4):
        shift = 8 * p

        def c_src(r):
            return (keys[r][1], keys[r][0], keys[r][1], keys[r][0])[p]

        def c_dst(r):
            return (keys[r][0], keys[r][1], keys[r][0], dataf[r])[p]

        def hmask(ks, sh):
            if sh >= 4:
                t = [lax.shift_right_logical(k, sh - 4) for k in ks]
            else:
                t = [lax.shift_left(k, 4) for k in ks]
            return [t_ & 0xFF0 for t_ in t]

        if p == 0:
            def hist_body(i, c):
                l_s = lax.shift_right_logical(i, 3)
                w_s = i & 7
                bases = [l_s * CHUNK + u * JB + w_s * L for u in range(U)]
                vs = [dataf[r][pl.ds(bases[u], L)] for (r, u) in SU]
                bs = [plsc.bitcast(v, jnp.int32) for v in vs]
                ms = [lax.shift_right_arithmetic(b, 31) for b in bs]
                ms = [m | INT_MIN for m in ms]
                ks = [b ^ m for b, m in zip(bs, ms)]
                for (r, u), k in zip(SU, ks):
                    keys[r][1][pl.ds(bases[u] + l_s, L)] = k
                hs = hmask(ks, 0)
                hidxs = [h | l_s for h in hs]
                for (r, u), h in zip(SU, hidxs):
                    plsc.addupdate_scatter(
                        hist[r], [h + np.int32(u * BINS * L)], ones)
                return c

            lax.fori_loop(0, CHUNK // U, hist_body, 0)

        def scan_body(i, carry):
            vs = [[hist[r][pl.ds((u * BINS + i) * L, L)] for u in range(U)]
                  for r in range(2)]
            for r in range(2):
                for u in range(U):
                    hist[r][pl.ds((u * BINS + i) * L, L)] = zeros
            t01 = [(v[0] + v[1], v[2] + v[3]) for v in vs]
            ts = [a + b for a, b in t01]
            css = [plsc.cumsum(t) for t in ts]
            excls = [cs - t + cry for cs, t, cry in zip(css, ts, carry)]
            tops = [jnp.take(cs, fifteen) for cs in css]
            nxt = tuple(cry + top for cry, top in zip(carry, tops))
            for r in range(2):
                acc = excls[r]
                for u in range(U):
                    offs[r][u][pl.ds(i * L, L)] = acc
                    if u < U - 1:
                        acc = acc + vs[r][u]
            return nxt

        lax.fori_loop(0, BINS, scan_body, (zvec, zvec))

        def scat_body(j, c):
            idxs = [lane9 + (u * JB + j) for u in range(U)]
            ks = [plsc.load_gather(c_src(r), [idxs[u]]) for (r, u) in SU]
            hs = hmask(ks, shift)
            hidxs = [h | lane for h in hs]
            poss = [plsc.load_gather(offs[r][u], [h])
                    for (r, u), h in zip(SU, hidxs)]
            if p == 3:
                ms = [lax.shift_right_arithmetic(k, 31) for k in ks]
                ms = [(~m) | INT_MIN for m in ms]
                outs = [plsc.bitcast(k ^ m, jnp.float32)
                        for k, m in zip(ks, ms)]
                st_poss = poss
            else:
                outs = ks
                lps = [lax.shift_right_logical(pos, 9) for pos in poss]
                st_poss = [pos + lp for pos, lp in zip(poss, lps)]
            for (r, u), pos, o in zip(SU, st_poss, outs):
                plsc.store_scatter(c_dst(r), [pos], o)
            for (r, u), h in zip(SU, hidxs):
                plsc.addupdate_scatter(offs[r][u], [h], ones)
            if p < 3:
                ups = [lax.shift_left(pos, 5) & 0x3000 for pos in poss]
                dps = hmask(ks, shift + 8)
                h1 = [a | b for a, b in zip(ups, lps)]
                h2 = [a | b for a, b in zip(h1, dps)]
                for (r, u), h in zip(SU, h2):
                    plsc.addupdate_scatter(hist[r], [h], ones)
            return c

        lax.fori_loop(0, JB, scat_body, 0)

    cout = [pltpu.async_copy(dataf[r], out_hbm.at[rows[r]], sems[r])
            for r in range(2)]
    for c in cout:
        c.wait()


_sc_sort = functools.partial(
    pl.kernel,
    out_type=jax.ShapeDtypeStruct((R, N), jnp.float32),
    mesh=plsc.VectorSubcoreMesh(core_axis_name="c", subcore_axis_name="s"),
    compiler_params=pltpu.CompilerParams(needs_layout_passes=False),
    scratch_types=[
        pltpu.VMEM((N,), jnp.float32),
        pltpu.VMEM((N,), jnp.float32),
        pltpu.VMEM((N + L,), jnp.int32),
        pltpu.VMEM((N + L,), jnp.int32),
        pltpu.VMEM((N + L,), jnp.int32),
        pltpu.VMEM((N + L,), jnp.int32),
        pltpu.VMEM((U * BINS * L,), jnp.int32),
        pltpu.VMEM((U * BINS * L,), jnp.int32),
    ] + [pltpu.VMEM((BINS * L,), jnp.int32) for _ in range(2 * U)]
      + [pltpu.SemaphoreType.DMA, pltpu.SemaphoreType.DMA],
)(_sort_body)


@jax.jit
def kernel(x):
    return _sc_sort(x)
